# baseline (speedup 1.0000x reference)
import jax
import jax.numpy as jnp
from jax.experimental import pallas as pl

NUM_CLASSES = 5
BLOCK = 16384


def _body(seq_ref, out_ref):
    s = seq_ref[:]
    classes = jax.lax.broadcasted_iota(jnp.int32, (NUM_CLASSES, BLOCK), 0)
    out_ref[:, :] = (s[None, :] == classes).astype(jnp.float32)


def kernel(seq):
    L = seq.shape[0]
    grid = pl.cdiv(L, BLOCK)
    return pl.pallas_call(
        _body,
        grid=(grid,),
        in_specs=[pl.BlockSpec((BLOCK,), lambda i: (i,))],
        out_specs=pl.BlockSpec((NUM_CLASSES, BLOCK), lambda i: (0, i)),
        out_shape=jax.ShapeDtypeStruct((NUM_CLASSES, L), jnp.float32),
    )(seq)
